# baseline (speedup 1.0000x reference)
import jax
import jax.numpy as jnp
from jax.experimental import pallas as pl
from jax.experimental.pallas import tpu as pltpu

_H = 1000
_R = 512


def _body(x_ref, o_ref):
    idx = x_ref[0, 0, :]
    iota = jax.lax.broadcasted_iota(jnp.int32, (_R, _H), 1)
    o_ref[...] = (idx[:, None] == iota).astype(jnp.float32)


def kernel(x):
    b, s = x.shape
    n = b * s
    g = n // _R
    xr = x.reshape(g, 1, _R).astype(jnp.int32)
    out = pl.pallas_call(
        _body,
        grid=(g,),
        in_specs=[pl.BlockSpec((1, 1, _R), lambda i: (i, 0, 0))],
        out_specs=pl.BlockSpec((_R, _H), lambda i: (i, 0)),
        out_shape=jax.ShapeDtypeStruct((n, _H), jnp.float32),
    )(xr)
    return out.reshape(b, s, _H)


# speedup vs baseline: 1.3301x; 1.3301x over previous
import jax
import jax.numpy as jnp
from jax.experimental import pallas as pl
from jax.experimental.pallas import tpu as pltpu

_H = 1000
_B = 16


def _body(x_ref, o_ref):
    idx = x_ref[...]
    s = idx.shape[1]
    iota = jax.lax.broadcasted_iota(jnp.int32, (_B, s, _H), 2)
    o_ref[...] = (idx[:, :, None] == iota).astype(jnp.float32)


def kernel(x):
    b, s = x.shape
    g = b // _B
    out = pl.pallas_call(
        _body,
        grid=(g,),
        in_specs=[pl.BlockSpec((_B, s), lambda i: (i, 0))],
        out_specs=pl.BlockSpec((_B, s, _H), lambda i: (i, 0, 0)),
        out_shape=jax.ShapeDtypeStruct((b, s, _H), jnp.float32),
    )(x.astype(jnp.int32))
    return out


# speedup vs baseline: 6.6529x; 5.0018x over previous
import jax
import jax.numpy as jnp
from jax.experimental import pallas as pl
from jax.experimental.pallas import tpu as pltpu

_H = 1000
_CC = 200


def _body(x_ref, o_ref):
    idx = x_ref[0, 0, :]
    b = idx.shape[0]
    c0 = pl.program_id(1) * _CC
    iota = c0 + jax.lax.broadcasted_iota(jnp.int32, (_CC, b), 0)
    o_ref[0] = (idx[None, :] == iota).astype(jnp.float32)


def kernel(x):
    b, s = x.shape
    xt = x.T.reshape(s, 1, b).astype(jnp.int32)
    out = pl.pallas_call(
        _body,
        grid=(s, _H // _CC),
        in_specs=[pl.BlockSpec((1, 1, b), lambda j, c: (j, 0, 0))],
        out_specs=pl.BlockSpec((1, _CC, b), lambda j, c: (j, c, 0)),
        out_shape=jax.ShapeDtypeStruct((s, _H, b), jnp.float32),
    )(xt)
    return jnp.transpose(out, (2, 0, 1))


# speedup vs baseline: 6.7652x; 1.0169x over previous
import jax
import jax.numpy as jnp
from jax.experimental import pallas as pl
from jax.experimental.pallas import tpu as pltpu

_H = 1000
_CC = 1000


def _body(x_ref, o_ref):
    idx = x_ref[0, 0, :]
    b = idx.shape[0]
    c0 = pl.program_id(1) * _CC
    iota = c0 + jax.lax.broadcasted_iota(jnp.int32, (_CC, b), 0)
    o_ref[0] = (idx[None, :] == iota).astype(jnp.float32)


def kernel(x):
    b, s = x.shape
    xt = x.T.reshape(s, 1, b).astype(jnp.int32)
    out = pl.pallas_call(
        _body,
        grid=(s, _H // _CC),
        in_specs=[pl.BlockSpec((1, 1, b), lambda j, c: (j, 0, 0))],
        out_specs=pl.BlockSpec((1, _CC, b), lambda j, c: (j, c, 0)),
        out_shape=jax.ShapeDtypeStruct((s, _H, b), jnp.float32),
    )(xt)
    return jnp.transpose(out, (2, 0, 1))
